# baseline (speedup 1.0000x reference)
import jax
import jax.numpy as jnp
from jax import lax
from jax.experimental import pallas as pl
from jax.experimental.pallas import tpu as pltpu, tpu_sc as plsc

N_NODES = 10000
N_EDGES = 320000
D = 128
RW = 0.001

NC = 2
NS = 16
NW = NC * NS

CH = 40
T = 250
NCHUNK = NW * T
E_PAD = NCHUNK * CH
RB = 8
IB = 8
LG = 4
LI = 5
N_PAD = 10240
ROWS_PER_SUBCORE = N_PAD // NS


def _sc_body(x_hbm, src_hbm, dst_hbm, part_hbm, agg, idx_s, idx_d, rows,
             gsem, ssem, isem):
    cid = lax.axis_index("c")
    sid = lax.axis_index("s")
    wid = sid * NC + cid
    cbase = wid * T

    def zrow(i, _):
        for j in range(D // 16):
            rows[0, i, pl.ds(j * 16, 16)] = jnp.zeros((16,), jnp.float32)
        return 0

    lax.fori_loop(0, CH, zrow, 0)

    zbase = sid * ROWS_PER_SUBCORE
    for k in range(ROWS_PER_SUBCORE // CH):
        pltpu.sync_copy(rows.at[0], agg.at[pl.ds(zbase + k * CH, CH)])
    plsc.subcore_barrier()

    def fire_i(j, bi):
        pltpu.async_copy(src_hbm.at[cbase + j], idx_s.at[bi], isem.at[bi])
        pltpu.async_copy(dst_hbm.at[cbase + j], idx_d.at[bi], isem.at[bi])

    def wait_i(j, bi):
        pltpu.make_async_copy(src_hbm.at[cbase + j], idx_s.at[bi], isem.at[bi]).wait()
        pltpu.make_async_copy(dst_hbm.at[cbase + j], idx_d.at[bi], isem.at[bi]).wait()

    def fire_g(j, bi, br):
        pltpu.async_copy(x_hbm.at[idx_s.at[bi]], rows.at[br], gsem.at[br])

    def wait_g(j, bi, br):
        pltpu.make_async_copy(x_hbm.at[idx_s.at[bi]], rows.at[br], gsem.at[br]).wait()

    def fire_s(j, bi, br):
        pltpu.async_copy(rows.at[br], agg.at[idx_d.at[bi]], ssem.at[br], add=True)

    def wait_s(j, bi, br):
        pltpu.make_async_copy(rows.at[br], agg.at[idx_d.at[bi]], ssem.at[br]).wait()

    def visit(v, mr, mi, do_ws, do_fg, do_fi):
        if do_ws:
            wait_s(v - 2, (mi + IB - 2) % IB, (mr + RB - 2) % RB)
        if do_fg:
            wait_i(v + LG, (mi + LG) % IB)
            fire_g(v + LG, (mi + LG) % IB, (mr + LG) % RB)
        if do_fi:
            fire_i(v + LI, (mi + LI) % IB)
        wait_g(v, mi, mr)
        fire_s(v, mi, mr)

    for j in range(LI):
        fire_i(j, j % IB)
    for j in range(LG):
        wait_i(j, j % IB)
        fire_g(j, j % IB, j % RB)
    for v in range(LG):
        visit(v, v % RB, v % IB, v >= 2, True, True)

    def group(g, _):
        v0 = LG + g * 8
        for i in range(8):
            visit(v0 + i, (LG + i) % RB, (LG + i) % IB, True, True, True)
        return 0

    NG = (T - 6 - LG) // 8
    lax.fori_loop(0, NG, group, 0)
    for v in range(LG + 8 * NG, T - 6):
        visit(v, v % RB, v % IB, True, True, True)

    for v in range(T - 6, T):
        visit(v, v % RB, v % IB, True, v + LG <= T - 1, v + LI <= T - 1)
    for v in range(T - 2, T):
        wait_s(v, v % IB, v % RB)

    plsc.subcore_barrier()

    pltpu.sync_copy(
        agg.at[pl.ds(zbase, ROWS_PER_SUBCORE)],
        part_hbm.at[cid, pl.ds(zbase, ROWS_PER_SUBCORE), :],
    )


_sc_segsum = pl.kernel(
    _sc_body,
    out_type=jax.ShapeDtypeStruct((NC, N_PAD, D), jnp.float32),
    mesh=plsc.VectorSubcoreMesh(
        core_axis_name="c", subcore_axis_name="s", num_cores=NC, num_subcores=NS
    ),
    scratch_types=[
        pltpu.VMEM_SHARED((N_PAD, D), jnp.float32),
        pltpu.VMEM((IB, CH), jnp.int32),
        pltpu.VMEM((IB, CH), jnp.int32),
        pltpu.VMEM((RB, CH, D), jnp.float32),
        pltpu.SemaphoreType.DMA((RB,)),
        pltpu.SemaphoreType.DMA((RB,)),
        pltpu.SemaphoreType.DMA((IB,)),
    ],
)


ROWS_TC = 1000


def _tc_body(p0_ref, x_ref, wl_ref, wc_ref, b_ref, o_ref):
    agg = p0_ref[0] + p0_ref[1]
    o_ref[...] = (
        jnp.dot(agg, wl_ref[...], preferred_element_type=jnp.float32)
        + jnp.dot(x_ref[...], wc_ref[...], preferred_element_type=jnp.float32)
        + b_ref[...]
    )


_tc_fused = pl.pallas_call(
    _tc_body,
    grid=(N_NODES // ROWS_TC,),
    in_specs=[
        pl.BlockSpec((NC, ROWS_TC, D), lambda i: (0, i, 0)),
        pl.BlockSpec((ROWS_TC, D), lambda i: (i, 0)),
        pl.BlockSpec((D, D), lambda i: (0, 0)),
        pl.BlockSpec((D, D), lambda i: (0, 0)),
        pl.BlockSpec((1, D), lambda i: (0, 0)),
    ],
    out_specs=pl.BlockSpec((ROWS_TC, D), lambda i: (i, 0)),
    out_shape=jax.ShapeDtypeStruct((N_NODES, D), jnp.float32),
)


def kernel(x, edge_index, W_l, b_l, W_r, W_res, b_res):
    src_p = edge_index[0].reshape(NCHUNK, CH)
    dst_p = edge_index[1].reshape(NCHUNK, CH)
    part0 = _sc_segsum(x, src_p, dst_p)

    wl_t = (RW * W_l).T
    wc_t = (RW * W_r + (1.0 - RW) * W_res).T
    bc = (RW * b_l + (1.0 - RW) * b_res).reshape(1, D)
    return _tc_fused(part0, x, wl_t, wc_t, bc)


# speedup vs baseline: 1.3636x; 1.3636x over previous
import jax
import jax.numpy as jnp
from jax import lax
from jax.experimental import pallas as pl
from jax.experimental.pallas import tpu as pltpu, tpu_sc as plsc

N_NODES = 10000
N_EDGES = 320000
D = 128
RW = 0.001

NC = 2
NS = 16
NW = NC * NS

CH = 80
T = 125
NCHUNK = NW * T
E_PAD = NCHUNK * CH
RB = 4
IB = 8
N_PAD = 10240
ROWS_PER_SUBCORE = N_PAD // NS


def _sc_body(x_hbm, src_hbm, dst_hbm, part_hbm, agg, idx_s, idx_d, rows,
             gsem, ssem, isem):
    cid = lax.axis_index("c")
    sid = lax.axis_index("s")
    wid = sid * NC + cid
    cbase = wid * T

    def zrow(i, _):
        for j in range(D // 16):
            rows[0, i, pl.ds(j * 16, 16)] = jnp.zeros((16,), jnp.float32)
        return 0

    lax.fori_loop(0, CH, zrow, 0)

    zbase = sid * ROWS_PER_SUBCORE
    for k in range(ROWS_PER_SUBCORE // CH):
        pltpu.sync_copy(rows.at[0], agg.at[pl.ds(zbase + k * CH, CH)])
    plsc.subcore_barrier()

    def fire_i(j, bi):
        pltpu.async_copy(src_hbm.at[cbase + j], idx_s.at[bi], isem.at[bi])
        pltpu.async_copy(dst_hbm.at[cbase + j], idx_d.at[bi], isem.at[bi])

    def wait_i(j, bi):
        pltpu.make_async_copy(src_hbm.at[cbase + j], idx_s.at[bi], isem.at[bi]).wait()
        pltpu.make_async_copy(dst_hbm.at[cbase + j], idx_d.at[bi], isem.at[bi]).wait()

    def fire_g(j, bi, br):
        pltpu.async_copy(x_hbm.at[idx_s.at[bi]], rows.at[br], gsem.at[br])

    def wait_g(j, bi, br):
        pltpu.make_async_copy(x_hbm.at[idx_s.at[bi]], rows.at[br], gsem.at[br]).wait()

    def fire_s(j, bi, br):
        pltpu.async_copy(rows.at[br], agg.at[idx_d.at[bi]], ssem.at[br], add=True)

    def wait_s(j, bi, br):
        pltpu.make_async_copy(rows.at[br], agg.at[idx_d.at[bi]], ssem.at[br]).wait()

    def visit(v, m4, m8, do_ws, do_fg, do_fi):
        if do_ws:
            wait_s(v - 2, (m8 + IB - 2) % IB, (m4 + 2) % RB)
        if do_fg:
            wait_i(v + 2, (m8 + 2) % IB)
            fire_g(v + 2, (m8 + 2) % IB, (m4 + 2) % RB)
        if do_fi:
            fire_i(v + 6, (m8 + 6) % IB)
        wait_g(v, m8, m4)
        fire_s(v, m8, m4)

    for j in range(6):
        fire_i(j, j)
    for j in range(2):
        wait_i(j, j)
        fire_g(j, j, j)
    for v in range(2):
        visit(v, v % RB, v % IB, False, True, True)

    def group(g, _):
        v0 = 2 + g * 8
        for i in range(8):
            visit(v0 + i, (2 + i) % RB, (2 + i) % IB, True, True, True)
        return 0

    NG = (T - 8) // 8
    lax.fori_loop(0, NG, group, 0)
    for v in range(2 + 8 * NG, T - 6):
        visit(v, v % RB, v % IB, True, True, True)

    for v in range(T - 6, T):
        visit(v, v % RB, v % IB, True, v + 2 <= T - 1, v + 6 <= T - 1)
    for v in range(T - 2, T):
        wait_s(v, v % IB, v % RB)

    plsc.subcore_barrier()

    pltpu.sync_copy(
        agg.at[pl.ds(zbase, ROWS_PER_SUBCORE)],
        part_hbm.at[cid, pl.ds(zbase, ROWS_PER_SUBCORE), :],
    )


_sc_segsum = pl.kernel(
    _sc_body,
    out_type=jax.ShapeDtypeStruct((NC, N_PAD, D), jnp.float32),
    mesh=plsc.VectorSubcoreMesh(
        core_axis_name="c", subcore_axis_name="s", num_cores=NC, num_subcores=NS
    ),
    scratch_types=[
        pltpu.VMEM_SHARED((N_PAD, D), jnp.float32),
        pltpu.VMEM((IB, CH), jnp.int32),
        pltpu.VMEM((IB, CH), jnp.int32),
        pltpu.VMEM((RB, CH, D), jnp.float32),
        pltpu.SemaphoreType.DMA((RB,)),
        pltpu.SemaphoreType.DMA((RB,)),
        pltpu.SemaphoreType.DMA((IB,)),
    ],
)


ROWS_TC = 1000


def _tc_body(p0_ref, x_ref, wl_ref, wr_ref, wres_ref, bl_ref, bres_ref, o_ref):
    agg = p0_ref[0] + p0_ref[1]
    wl = RW * wl_ref[...]
    wc = RW * wr_ref[...] + (1.0 - RW) * wres_ref[...]
    bc = RW * bl_ref[...] + (1.0 - RW) * bres_ref[...]
    o_ref[...] = (
        jax.lax.dot_general(
            agg, wl, (((1,), (1,)), ((), ())), preferred_element_type=jnp.float32
        )
        + jax.lax.dot_general(
            x_ref[...], wc, (((1,), (1,)), ((), ())),
            preferred_element_type=jnp.float32,
        )
        + bc
    )


_tc_fused = pl.pallas_call(
    _tc_body,
    grid=(N_NODES // ROWS_TC,),
    in_specs=[
        pl.BlockSpec((NC, ROWS_TC, D), lambda i: (0, i, 0)),
        pl.BlockSpec((ROWS_TC, D), lambda i: (i, 0)),
        pl.BlockSpec((D, D), lambda i: (0, 0)),
        pl.BlockSpec((D, D), lambda i: (0, 0)),
        pl.BlockSpec((D, D), lambda i: (0, 0)),
        pl.BlockSpec((1, D), lambda i: (0, 0)),
        pl.BlockSpec((1, D), lambda i: (0, 0)),
    ],
    out_specs=pl.BlockSpec((ROWS_TC, D), lambda i: (i, 0)),
    out_shape=jax.ShapeDtypeStruct((N_NODES, D), jnp.float32),
)


def kernel(x, edge_index, W_l, b_l, W_r, W_res, b_res):
    src_p = edge_index[0].reshape(NCHUNK, CH)
    dst_p = edge_index[1].reshape(NCHUNK, CH)
    part0 = _sc_segsum(x, src_p, dst_p)

    return _tc_fused(
        part0, x, W_l, W_r, W_res, b_l.reshape(1, D), b_res.reshape(1, D)
    )
